# baseline (speedup 1.0000x reference)
import jax
import jax.numpy as jnp
from jax import lax
from jax.experimental import pallas as pl
from jax.experimental.pallas import tpu as pltpu
from jax.experimental.pallas import tpu_sc as plsc

VOCAB = 1000000
EMBED = 64
BATCH = 16384
IDX_MINOR = 128
L = 16
NW = 32
BPW = BATCH // NW
NCH = BPW // IDX_MINOR
CHUNK = 65536


def _mm_kernel(t_blk, w_blk, p0, p1, p2, p3):
    x = t_blk[...]
    w = w_blk[...]
    y = lax.dot_general(w, x, (((0,), (0,)), ((), ())),
                        preferred_element_type=jnp.float32)
    p0[...] = y[0, :]
    p1[...] = y[1, :]
    p2[...] = y[2, :]
    p3[...] = y[3, :]


def _sc_kernel(p0, p1, p2, p3, h2, t2, bsp, out0, out1,
               hidx, tidx, g0, g1, g2, g3, bv, o0v, o1v, sem):
    wid = lax.axis_index("s") * 2 + lax.axis_index("c")

    pltpu.sync_copy(h2.at[pl.ds(wid * NCH, NCH)], hidx)
    pltpu.sync_copy(t2.at[pl.ds(wid * NCH, NCH)], tidx)
    pltpu.sync_copy(bsp, bv)

    copies = []
    for k in range(NCH):
        sl = pl.ds(IDX_MINOR * k, IDX_MINOR)
        copies.append(pltpu.async_copy(p0.at[hidx.at[k]], g0.at[sl], sem))
        copies.append(pltpu.async_copy(p1.at[hidx.at[k]], g1.at[sl], sem))
        copies.append(pltpu.async_copy(p2.at[tidx.at[k]], g2.at[sl], sem))
        copies.append(pltpu.async_copy(p3.at[tidx.at[k]], g3.at[sl], sem))
    for c in copies:
        c.wait()

    b0 = bv[0, pl.ds(0, L)]
    b1 = bv[1, pl.ds(0, L)]

    def body(sv, carry):
        sl = pl.ds(sv * L, L)
        o0v[sl] = g0[sl] + g2[sl] + b0
        o1v[sl] = g1[sl] + g3[sl] + b1
        return carry

    lax.fori_loop(0, BPW // L, body, 0)

    pltpu.sync_copy(o0v, out0.at[pl.ds(wid * BPW, BPW)])
    pltpu.sync_copy(o1v, out1.at[pl.ds(wid * BPW, BPW)])


def kernel(head, tail, table, fc_w, fc_b):
    tT = table.T
    w8 = jnp.zeros((EMBED, 8), jnp.float32)
    w8 = w8.at[:, 0].set(fc_w[:EMBED, 0]).at[:, 1].set(fc_w[:EMBED, 1])
    w8 = w8.at[:, 2].set(fc_w[EMBED:, 0]).at[:, 3].set(fc_w[EMBED:, 1])

    grid = pl.cdiv(VOCAB, CHUNK)
    planes = pl.pallas_call(
        _mm_kernel,
        grid=(grid,),
        in_specs=[
            pl.BlockSpec((EMBED, CHUNK), lambda i: (0, i)),
            pl.BlockSpec((EMBED, 8), lambda i: (0, 0)),
        ],
        out_specs=[pl.BlockSpec((CHUNK,), lambda i: (i,))] * 4,
        out_shape=[jax.ShapeDtypeStruct((VOCAB,), jnp.float32)] * 4,
    )(tT, w8)
    p0, p1, p2, p3 = planes

    h2 = head.reshape(BATCH // IDX_MINOR, IDX_MINOR)
    t2 = tail.reshape(BATCH // IDX_MINOR, IDX_MINOR)
    bsp = jnp.broadcast_to(fc_b[:, None], (2, L))

    mesh = plsc.VectorSubcoreMesh(core_axis_name="c", subcore_axis_name="s")
    run = pl.kernel(
        _sc_kernel,
        mesh=mesh,
        compiler_params=pltpu.CompilerParams(
            needs_layout_passes=False, use_tc_tiling_on_sc=False),
        out_type=[
            jax.ShapeDtypeStruct((BATCH,), jnp.float32),
            jax.ShapeDtypeStruct((BATCH,), jnp.float32),
        ],
        scratch_types=[
            pltpu.VMEM((NCH, IDX_MINOR), jnp.int32),
            pltpu.VMEM((NCH, IDX_MINOR), jnp.int32),
            pltpu.VMEM((BPW,), jnp.float32),
            pltpu.VMEM((BPW,), jnp.float32),
            pltpu.VMEM((BPW,), jnp.float32),
            pltpu.VMEM((BPW,), jnp.float32),
            pltpu.VMEM((2, L), jnp.float32),
            pltpu.VMEM((BPW,), jnp.float32),
            pltpu.VMEM((BPW,), jnp.float32),
            pltpu.SemaphoreType.DMA,
        ],
    )
    o0, o1 = run(p0, p1, p2, p3, h2, t2, bsp)
    return jnp.stack([o0, o1], axis=1)


# speedup vs baseline: 1.0004x; 1.0004x over previous
import jax
import jax.numpy as jnp
from jax import lax
from jax.experimental import pallas as pl
from jax.experimental.pallas import tpu as pltpu
from jax.experimental.pallas import tpu_sc as plsc

VOCAB = 1000000
EMBED = 64
BATCH = 16384
IDX_MINOR = 128
L = 16
NW = 32
BPW = BATCH // NW
NCH = BPW // IDX_MINOR
CHUNK = 32768


def _mm_kernel(t_blk, w_blk, p0, p1, p2, p3):
    x = t_blk[...]
    w = w_blk[...]
    y = lax.dot_general(w, x, (((0,), (0,)), ((), ())),
                        preferred_element_type=jnp.float32)
    p0[...] = y[0, :]
    p1[...] = y[1, :]
    p2[...] = y[2, :]
    p3[...] = y[3, :]


def _sc_kernel(p0, p1, p2, p3, h2, t2, bsp, out0, out1,
               hidx, tidx, g0, g1, g2, g3, bv, o0v, o1v, sem):
    wid = lax.axis_index("s") * 2 + lax.axis_index("c")

    pltpu.sync_copy(h2.at[pl.ds(wid * NCH, NCH)], hidx)
    pltpu.sync_copy(t2.at[pl.ds(wid * NCH, NCH)], tidx)
    pltpu.sync_copy(bsp, bv)

    copies = []
    for k in range(NCH):
        sl = pl.ds(IDX_MINOR * k, IDX_MINOR)
        copies.append(pltpu.async_copy(p0.at[hidx.at[k]], g0.at[sl], sem))
        copies.append(pltpu.async_copy(p1.at[hidx.at[k]], g1.at[sl], sem))
        copies.append(pltpu.async_copy(p2.at[tidx.at[k]], g2.at[sl], sem))
        copies.append(pltpu.async_copy(p3.at[tidx.at[k]], g3.at[sl], sem))
    for c in copies:
        c.wait()

    b0 = bv[0, pl.ds(0, L)]
    b1 = bv[1, pl.ds(0, L)]

    def body(sv, carry):
        sl = pl.ds(sv * L, L)
        o0v[sl] = g0[sl] + g2[sl] + b0
        o1v[sl] = g1[sl] + g3[sl] + b1
        return carry

    lax.fori_loop(0, BPW // L, body, 0)

    pltpu.sync_copy(o0v, out0.at[pl.ds(wid * BPW, BPW)])
    pltpu.sync_copy(o1v, out1.at[pl.ds(wid * BPW, BPW)])


def kernel(head, tail, table, fc_w, fc_b):
    tT = table.T
    w8 = jnp.zeros((EMBED, 8), jnp.float32)
    w8 = w8.at[:, 0].set(fc_w[:EMBED, 0]).at[:, 1].set(fc_w[:EMBED, 1])
    w8 = w8.at[:, 2].set(fc_w[EMBED:, 0]).at[:, 3].set(fc_w[EMBED:, 1])

    grid = pl.cdiv(VOCAB, CHUNK)
    planes = pl.pallas_call(
        _mm_kernel,
        grid=(grid,),
        in_specs=[
            pl.BlockSpec((EMBED, CHUNK), lambda i: (0, i)),
            pl.BlockSpec((EMBED, 8), lambda i: (0, 0)),
        ],
        out_specs=[pl.BlockSpec((CHUNK,), lambda i: (i,))] * 4,
        out_shape=[jax.ShapeDtypeStruct((VOCAB,), jnp.float32)] * 4,
    )(tT, w8)
    p0, p1, p2, p3 = planes

    h2 = head.reshape(BATCH // IDX_MINOR, IDX_MINOR)
    t2 = tail.reshape(BATCH // IDX_MINOR, IDX_MINOR)
    bsp = jnp.broadcast_to(fc_b[:, None], (2, L))

    mesh = plsc.VectorSubcoreMesh(core_axis_name="c", subcore_axis_name="s")
    run = pl.kernel(
        _sc_kernel,
        mesh=mesh,
        compiler_params=pltpu.CompilerParams(
            needs_layout_passes=False, use_tc_tiling_on_sc=False),
        out_type=[
            jax.ShapeDtypeStruct((BATCH,), jnp.float32),
            jax.ShapeDtypeStruct((BATCH,), jnp.float32),
        ],
        scratch_types=[
            pltpu.VMEM((NCH, IDX_MINOR), jnp.int32),
            pltpu.VMEM((NCH, IDX_MINOR), jnp.int32),
            pltpu.VMEM((BPW,), jnp.float32),
            pltpu.VMEM((BPW,), jnp.float32),
            pltpu.VMEM((BPW,), jnp.float32),
            pltpu.VMEM((BPW,), jnp.float32),
            pltpu.VMEM((2, L), jnp.float32),
            pltpu.VMEM((BPW,), jnp.float32),
            pltpu.VMEM((BPW,), jnp.float32),
            pltpu.SemaphoreType.DMA,
        ],
    )
    o0, o1 = run(p0, p1, p2, p3, h2, t2, bsp)
    return jnp.stack([o0, o1], axis=1)
